# initial kernel scaffold (unmeasured)
import jax
import jax.numpy as jnp
from jax import lax
from jax.experimental import pallas as pl
from jax.experimental.pallas import tpu as pltpu

N_DEV = 4


def _a2a_body(y_ref, out_ref, recv_buf, send_sems, recv_sems):
    my = lax.axis_index("i")
    m_per, n_full = y_ref.shape
    n_per = n_full // N_DEV

    barrier = pltpu.get_barrier_semaphore()
    for k in range(1, N_DEV):
        pl.semaphore_signal(
            barrier, inc=1,
            device_id=((my + k) % N_DEV,),
            device_id_type=pl.DeviceIdType.MESH,
        )
    pl.semaphore_wait(barrier, N_DEV - 1)

    sends = []
    for k in range(1, N_DEV):
        tgt = (my + k) % N_DEV
        rdma = pltpu.make_async_remote_copy(
            src_ref=y_ref.at[:, pl.ds(tgt * n_per, n_per)],
            dst_ref=recv_buf.at[my],
            send_sem=send_sems.at[k - 1],
            recv_sem=recv_sems.at[my],
            device_id=(tgt,),
            device_id_type=pl.DeviceIdType.MESH,
        )
        rdma.start()
        sends.append(rdma)

    out_ref[pl.ds(my * m_per, m_per), :] = (
        y_ref[:, pl.ds(my * n_per, n_per)].astype(jnp.float32)
    )

    for k in range(1, N_DEV):
        src_dev = (my + k) % N_DEV
        recv = pltpu.make_async_remote_copy(
            src_ref=y_ref.at[:, pl.ds(0, n_per)],
            dst_ref=recv_buf.at[src_dev],
            send_sem=send_sems.at[0],
            recv_sem=recv_sems.at[src_dev],
            device_id=(src_dev,),
            device_id_type=pl.DeviceIdType.MESH,
        )
        recv.wait_recv()
        out_ref[pl.ds(src_dev * m_per, m_per), :] = (
            recv_buf[src_dev].astype(jnp.float32)
        )

    for rdma in sends:
        rdma.wait_send()


def kernel(x, w_mat, scale_x, scale_w):
    xb = x.astype(jnp.bfloat16)
    wb = w_mat.astype(jnp.bfloat16)
    acc = jnp.dot(xb, wb, preferred_element_type=jnp.float32)
    y = acc * (scale_x[0] * scale_w[0])
    y = y * jax.nn.sigmoid(y)
    y16 = y.astype(jnp.bfloat16)

    m_per, n_full = y16.shape
    n_per = n_full // N_DEV
    return pl.pallas_call(
        _a2a_body,
        out_shape=jax.ShapeDtypeStruct((N_DEV * m_per, n_per), jnp.float32),
        in_specs=[pl.BlockSpec(memory_space=pltpu.VMEM)],
        out_specs=pl.BlockSpec(memory_space=pltpu.VMEM),
        scratch_shapes=[
            pltpu.VMEM((N_DEV, m_per, n_per), jnp.bfloat16),
            pltpu.SemaphoreType.DMA((N_DEV - 1,)),
            pltpu.SemaphoreType.DMA((N_DEV,)),
        ],
        compiler_params=pltpu.CompilerParams(collective_id=0),
    )(y16)


# baseline (device time: 55323 ns/iter reference)
import jax
import jax.numpy as jnp
from jax import lax
from jax.experimental import pallas as pl
from jax.experimental.pallas import tpu as pltpu

N_DEV = 4


def _body(s_ref, x_ref, w_ref, out_ref, ybuf, recv_buf, send_sems, recv_sems):
    my = lax.axis_index("i")
    m_per = x_ref.shape[0]
    n_per = out_ref.shape[1]

    barrier = pltpu.get_barrier_semaphore()
    for k in range(1, N_DEV):
        pl.semaphore_signal(
            barrier, inc=1,
            device_id=((my + k) % N_DEV,),
            device_id_type=pl.DeviceIdType.MESH,
        )
    pl.semaphore_wait(barrier, N_DEV - 1)

    s = s_ref[0]

    sends = []
    for k in range(1, N_DEV):
        tgt = (my + k) % N_DEV
        wblk = w_ref[:, pl.ds(tgt * n_per, n_per)]
        acc = jnp.dot(x_ref[...], wblk, preferred_element_type=jnp.float32)
        y = acc * s
        y = y * jax.nn.sigmoid(y)
        ybuf[k - 1] = y.astype(jnp.bfloat16)
        rdma = pltpu.make_async_remote_copy(
            src_ref=ybuf.at[k - 1],
            dst_ref=recv_buf.at[3 - k],
            send_sem=send_sems.at[k - 1],
            recv_sem=recv_sems.at[3 - k],
            device_id=(tgt,),
            device_id_type=pl.DeviceIdType.MESH,
        )
        rdma.start()
        sends.append(rdma)

    wblk = w_ref[:, pl.ds(my * n_per, n_per)]
    acc = jnp.dot(x_ref[...], wblk, preferred_element_type=jnp.float32)
    y = acc * s
    out_ref[pl.ds(my * m_per, m_per), :] = y * jax.nn.sigmoid(y)

    for k in range(N_DEV - 1, 0, -1):
        src_dev = (my + k) % N_DEV
        recv = pltpu.make_async_remote_copy(
            src_ref=ybuf.at[0],
            dst_ref=recv_buf.at[k - 1],
            send_sem=send_sems.at[0],
            recv_sem=recv_sems.at[k - 1],
            device_id=(src_dev,),
            device_id_type=pl.DeviceIdType.MESH,
        )
        recv.wait_recv()
        out_ref[pl.ds(src_dev * m_per, m_per), :] = (
            recv_buf[k - 1].astype(jnp.float32)
        )

    for rdma in sends:
        rdma.wait_send()


def kernel(x, w_mat, scale_x, scale_w):
    m_per = x.shape[0]
    n_full = w_mat.shape[1]
    n_per = n_full // N_DEV
    s = (scale_x[0] * scale_w[0]).reshape(1).astype(jnp.float32)

    x8 = x.astype(jnp.float8_e4m3fn)
    w8 = w_mat.astype(jnp.float8_e4m3fn)

    return pl.pallas_call(
        _body,
        out_shape=jax.ShapeDtypeStruct((N_DEV * m_per, n_per), jnp.float32),
        in_specs=[
            pl.BlockSpec(memory_space=pltpu.SMEM),
            pl.BlockSpec(memory_space=pltpu.VMEM),
            pl.BlockSpec(memory_space=pltpu.VMEM),
        ],
        out_specs=pl.BlockSpec(memory_space=pltpu.VMEM),
        scratch_shapes=[
            pltpu.VMEM((N_DEV - 1, m_per, n_per), jnp.bfloat16),
            pltpu.VMEM((N_DEV - 1, m_per, n_per), jnp.bfloat16),
            pltpu.SemaphoreType.DMA((N_DEV - 1,)),
            pltpu.SemaphoreType.DMA((N_DEV - 1,)),
        ],
        compiler_params=pltpu.CompilerParams(collective_id=0),
    )(s, x8, w8)
